# baseline (speedup 1.0000x reference)
import functools

import jax
import jax.numpy as jnp
from jax import lax
from jax.experimental import pallas as pl
from jax.experimental.pallas import tpu as pltpu
from jax.experimental.pallas import tpu_sc as plsc

R = 128
C = 32768
L = 16
NC = 2
NS = 16
NW = NC * NS
ROWS_PER_W = R // NW
BLK = 16
NBLK = C // (BLK * L)
BIG = jnp.int32(2**30)
TC_ROWS = 8

_mesh = plsc.VectorSubcoreMesh(core_axis_name="c", subcore_axis_name="s")


def _shuffle(x, idx):
    return lax.gather(
        x, idx[:, None],
        lax.GatherDimensionNumbers(
            offset_dims=(), collapsed_slice_dims=(0,), start_index_map=(0,)),
        slice_sizes=(1,),
        mode=lax.GatherScatterMode.PROMISE_IN_BOUNDS)


@functools.partial(
    pl.kernel,
    out_type=jax.ShapeDtypeStruct((NW, L), jnp.int32),
    mesh=_mesh,
    scratch_types=[
        pltpu.VMEM((C,), jnp.float32),
        pltpu.VMEM((C,), jnp.float32),
        pltpu.VMEM((L,), jnp.int32),
        pltpu.SemaphoreType.DMA,
        pltpu.SemaphoreType.DMA,
        pltpu.SemaphoreType.DMA,
    ],
    compiler_params=pltpu.CompilerParams(needs_layout_passes=False),
)
def _argmax_idx(data_hbm, idx_hbm, in0, in1, idx_st, sem0, sem1, sem_out):
    wid = lax.axis_index("s") * NC + lax.axis_index("c")
    lanes = lax.iota(jnp.int32, L)
    ones_i = jnp.ones((L,), jnp.int32)
    bufs = (in0, in1)
    sems = (sem0, sem1)
    base_row = wid * ROWS_PER_W

    cps = [pltpu.async_copy(data_hbm.at[base_row], in0, sem0), None]

    iv = jnp.zeros((L,), jnp.int32)
    for r in range(ROWS_PER_W):
        cps[r % 2].wait()
        if r + 1 < ROWS_PER_W:
            cps[(r + 1) % 2] = pltpu.async_copy(
                data_hbm.at[base_row + r + 1], bufs[(r + 1) % 2],
                sems[(r + 1) % 2])
        buf = bufs[r % 2]

        def block_step(j, carry, buf=buf):
            bv, bb, jv = carry
            bvp = bv
            base = j * (BLK * L)
            for k in range(BLK):
                v = buf[pl.ds(base + k * L, L)]
                bv = jnp.maximum(bv, v)
            upd = bv > bvp
            bb = jnp.where(upd, jv, bb)
            return bv, bb, jv + ones_i

        init = (jnp.full((L,), -jnp.inf, jnp.float32),
                jnp.zeros((L,), jnp.int32),
                jnp.zeros((L,), jnp.int32))
        bv, bb, _ = lax.fori_loop(0, NBLK, block_step, init)

        for k in (8, 4, 2, 1):
            pv = _shuffle(bv, lanes ^ k)
            pb = _shuffle(bb, lanes ^ k)
            take = (pv > bv) | ((pv == bv) & (pb < bb))
            bv = jnp.where(take, pv, bv)
            bb = jnp.where(take, pb, bb)

        base_idx = bb * (BLK * L) + lanes
        fi = jnp.full((L,), BIG, jnp.int32)
        for c in range(BLK):
            idx = base_idx + (c * L)
            v = plsc.load_gather(buf, [idx])
            fi = jnp.minimum(fi, jnp.where(v == bv, idx, BIG))
        for k in (8, 4, 2, 1):
            fi = jnp.minimum(fi, _shuffle(fi, lanes ^ k))

        iv = jnp.where(lanes == r, fi, iv)

    idx_st[pl.ds(0, L)] = iv
    pltpu.async_copy(idx_st, idx_hbm.at[wid], sem_out).wait()


def _onehot_body(idx_ref, out_ref):
    i = pl.program_id(0)
    col = lax.broadcasted_iota(jnp.int32, (TC_ROWS, C), 1)
    targets = []
    for j in range(TC_ROWS):
        r = i * TC_ROWS + j
        targets.append(idx_ref[r // ROWS_PER_W, r % ROWS_PER_W])
    tvec = jnp.stack(targets)[:, None]
    out_ref[:, :] = (col == tvec).astype(jnp.float32)


_onehot_tc = pl.pallas_call(
    _onehot_body,
    grid=(R // TC_ROWS,),
    in_specs=[pl.BlockSpec(memory_space=pltpu.SMEM)],
    out_specs=pl.BlockSpec((TC_ROWS, C), lambda i: (i, 0)),
    out_shape=jax.ShapeDtypeStruct((R, C), jnp.float32),
)


def kernel(data):
    return _onehot_tc(_argmax_idx(data))


# speedup vs baseline: 1.1875x; 1.1875x over previous
import functools

import jax
import jax.numpy as jnp
from jax import lax
from jax.experimental import pallas as pl
from jax.experimental.pallas import tpu as pltpu
from jax.experimental.pallas import tpu_sc as plsc

R = 128
C = 32768
L = 16
NC = 2
NS = 16
NW = NC * NS
ROWS_PER_W = R // NW
BLK = 16
NBLK = C // (BLK * L)
BIG = jnp.int32(2**30)

_mesh = plsc.VectorSubcoreMesh(core_axis_name="c", subcore_axis_name="s")


def _shuffle(x, idx):
    return lax.gather(
        x, idx[:, None],
        lax.GatherDimensionNumbers(
            offset_dims=(), collapsed_slice_dims=(0,), start_index_map=(0,)),
        slice_sizes=(1,),
        mode=lax.GatherScatterMode.PROMISE_IN_BOUNDS)


@functools.partial(
    pl.kernel,
    out_type=jax.ShapeDtypeStruct((R, C), jnp.float32),
    mesh=_mesh,
    scratch_types=[
        pltpu.VMEM((C,), jnp.float32),
        pltpu.VMEM((C,), jnp.float32),
        pltpu.VMEM((C,), jnp.float32),
        pltpu.SemaphoreType.DMA,
        pltpu.SemaphoreType.DMA,
        pltpu.SemaphoreType.DMA,
    ],
    compiler_params=pltpu.CompilerParams(needs_layout_passes=False),
)
def _argmax_onehot(data_hbm, out_hbm, in0, in1, out_v, sem0, sem1, sem_out):
    wid = lax.axis_index("s") * NC + lax.axis_index("c")
    lanes = lax.iota(jnp.int32, L)
    zeros = jnp.zeros((L,), jnp.float32)
    ones = jnp.ones((L,), jnp.float32)
    ones_i = jnp.ones((L,), jnp.int32)
    bufs = (in0, in1)
    sems = (sem0, sem1)
    base_row = wid * ROWS_PER_W

    cps = [pltpu.async_copy(data_hbm.at[base_row], in0, sem0), None]

    def zfill(t, _):
        base = t * (8 * L)
        for k in range(8):
            out_v[pl.ds(base + k * L, L)] = zeros
        return 0

    lax.fori_loop(0, C // (8 * L), zfill, 0)

    out_cp = None
    prev_bi = None
    for r in range(ROWS_PER_W):
        cps[r % 2].wait()
        if r + 1 < ROWS_PER_W:
            cps[(r + 1) % 2] = pltpu.async_copy(
                data_hbm.at[base_row + r + 1], bufs[(r + 1) % 2],
                sems[(r + 1) % 2])
        buf = bufs[r % 2]

        def block_step(j, carry, buf=buf):
            bv, bb, jv = carry
            bvp = bv
            base = j * (BLK * L)
            for k in range(BLK):
                v = buf[pl.ds(base + k * L, L)]
                bv = jnp.maximum(bv, v)
            upd = bv > bvp
            bb = jnp.where(upd, jv, bb)
            return bv, bb, jv + ones_i

        init = (jnp.full((L,), -jnp.inf, jnp.float32),
                jnp.zeros((L,), jnp.int32),
                jnp.zeros((L,), jnp.int32))
        bv, bb, _ = lax.fori_loop(0, NBLK, block_step, init)

        for k in (8, 4, 2, 1):
            pv = _shuffle(bv, lanes ^ k)
            pb = _shuffle(bb, lanes ^ k)
            take = (pv > bv) | ((pv == bv) & (pb < bb))
            bv = jnp.where(take, pv, bv)
            bb = jnp.where(take, pb, bb)

        base_idx = bb * (BLK * L) + lanes
        fi = jnp.full((L,), BIG, jnp.int32)
        for c in range(BLK):
            idx = base_idx + (c * L)
            v = plsc.load_gather(buf, [idx])
            fi = jnp.minimum(fi, jnp.where(v == bv, idx, BIG))
        for k in (8, 4, 2, 1):
            fi = jnp.minimum(fi, _shuffle(fi, lanes ^ k))

        if out_cp is not None:
            out_cp.wait()
            plsc.store_scatter(out_v, [prev_bi], zeros, mask=lanes == 0)
        plsc.store_scatter(out_v, [fi], ones, mask=lanes == 0)
        out_cp = pltpu.async_copy(out_v, out_hbm.at[base_row + r], sem_out)
        prev_bi = fi

    out_cp.wait()


def kernel(data):
    return _argmax_onehot(data)
